# baseline (speedup 1.0000x reference)
import jax
import jax.numpy as jnp
from jax.experimental import pallas as pl
from jax.experimental.pallas import tpu as pltpu

N = 4096
BM = 512


def _prep_kernel(x_ref, w1_ref, scale_ref, t_ref):
    t = jnp.dot(x_ref[...], w1_ref[...], preferred_element_type=jnp.float32)
    t_ref[...] = (t * scale_ref[...]).astype(jnp.bfloat16)


def _hidden_kernel(adj_ref, t_ref, c_ref, wa_ref, p_ref):
    h = jnp.dot(adj_ref[...].astype(jnp.bfloat16), t_ref[...],
                preferred_element_type=jnp.float32)
    h = jnp.maximum(h + c_ref[...], 0.0)
    p_ref[...] = jnp.dot(h.astype(jnp.bfloat16), wa_ref[...],
                         preferred_element_type=jnp.float32).astype(jnp.bfloat16)


def _out_kernel(adj_ref, p_ref, ba_ref, o_ref):
    logits = jnp.dot(adj_ref[...].astype(jnp.bfloat16), p_ref[...],
                     preferred_element_type=jnp.float32)
    logits = logits + ba_ref[...]
    m = jnp.max(logits, axis=1, keepdims=True)
    z = logits - m
    o_ref[...] = z - jnp.log(jnp.sum(jnp.exp(z), axis=1, keepdims=True))


_PARALLEL = pltpu.CompilerParams(dimension_semantics=("parallel",))


def kernel(x, adj, W1, b1, bn_gamma, bn_beta, bn_mean, bn_var, Wa, ba):
    nfeat = x.shape[1]
    nhid = W1.shape[1]
    nheads, _, nclass = Wa.shape
    ncat = nheads * nclass

    scale = bn_gamma / jnp.sqrt(bn_var + 1e-5)
    c = ((b1 - bn_mean) * scale + bn_beta).reshape(1, nhid)
    scale = scale.reshape(1, nhid)
    wa_cat = jnp.transpose(Wa, (1, 0, 2)).reshape(nhid, ncat).astype(jnp.bfloat16)
    ba_cat = ba.reshape(1, ncat)

    t = pl.pallas_call(
        _prep_kernel,
        out_shape=jax.ShapeDtypeStruct((N, nhid), jnp.bfloat16),
    )(x.astype(jnp.bfloat16), W1.astype(jnp.bfloat16), scale)

    nb = N // BM
    p = pl.pallas_call(
        _hidden_kernel,
        grid=(nb,),
        in_specs=[
            pl.BlockSpec((BM, N), lambda i: (i, 0)),
            pl.BlockSpec((N, nhid), lambda i: (0, 0)),
            pl.BlockSpec((1, nhid), lambda i: (0, 0)),
            pl.BlockSpec((nhid, ncat), lambda i: (0, 0)),
        ],
        out_specs=pl.BlockSpec((BM, ncat), lambda i: (i, 0)),
        out_shape=jax.ShapeDtypeStruct((N, ncat), jnp.bfloat16),
        compiler_params=_PARALLEL,
    )(adj, t, c, wa_cat)

    out = pl.pallas_call(
        _out_kernel,
        grid=(nb,),
        in_specs=[
            pl.BlockSpec((BM, N), lambda i: (i, 0)),
            pl.BlockSpec((N, ncat), lambda i: (0, 0)),
            pl.BlockSpec((1, ncat), lambda i: (0, 0)),
        ],
        out_specs=pl.BlockSpec((BM, ncat), lambda i: (i, 0)),
        out_shape=jax.ShapeDtypeStruct((N, ncat), jnp.float32),
        compiler_params=_PARALLEL,
    )(adj, p, ba_cat)
    return out
